# Initial kernel scaffold; baseline (speedup 1.0000x reference)
import jax
import jax.numpy as jnp
from jax.experimental import pallas as pl


def kernel(node_features, edge_index, edge_type, edge_attr, W1, att_src1, att_dst1, W_edge1, att_edge1, bias1, gamma1, beta1, W2, att_src2, att_dst2, W_edge2, att_edge2, bias2, gamma2, beta2, W_out, b_out):
    raise NotImplementedError("write your pallas kernel here")


# speedup vs baseline: 38.2450x; 38.2450x over previous
import functools

import jax
import jax.numpy as jnp
from jax import lax
from jax.experimental import pallas as pl
from jax.experimental.pallas import tpu as pltpu
from jax.experimental.pallas import tpu_sc as plsc

N = 50000
E = 800000
HEADS = 4
HD = 16
HID = 64

NC = 2
NS = 16
EPT = E // NS
CH = 128
NFULL = EPT // CH
TAIL = EPT - NFULL * CH
NPAD = 51200
ROWS_T = NPAD // NS
D_PT = 2 * NPAD // NS

BN = 2000
NB = N // BN


def _prep_body(x_ref, w_ref, asrc_ref, adst_ref, ea_ref, t_ref, q_ref,
               easum_ref):
    s = pl.program_id(0)
    i = pl.program_id(1)
    h = jnp.dot(x_ref[...], w_ref[...], preferred_element_type=jnp.float32)
    a_s = jnp.dot(h, asrc_ref[...], preferred_element_type=jnp.float32)
    a_d = jnp.dot(h, adst_ref[...], preferred_element_type=jnp.float32)
    t_ref[...] = jnp.where(s == 0, h[:, :32], h[:, 32:])
    q0 = jnp.concatenate([a_s[:, 0:2], a_d[:, 0:2]], axis=1)
    q1 = jnp.concatenate([a_s[:, 2:4], a_d[:, 2:4]], axis=1)
    q_ref[...] = jnp.where(s == 0, q0, q1)

    @pl.when((s == 0) & (i == 0))
    def _():
        easum_ref[...] = jnp.zeros_like(easum_ref)

    @pl.when(s == 0)
    def _():
        easum_ref[...] += jnp.sum(ea_ref[...])[None, None]


_prep_call = pl.pallas_call(
    _prep_body,
    grid=(2, NB),
    in_specs=[
        pl.BlockSpec((BN, HID), lambda s, i: (i, 0)),
        pl.BlockSpec((HID, HID), lambda s, i: (0, 0)),
        pl.BlockSpec((HID, HEADS), lambda s, i: (0, 0)),
        pl.BlockSpec((HID, HEADS), lambda s, i: (0, 0)),
        pl.BlockSpec((800 // NB, 1000), lambda s, i: (i, 0)),
    ],
    out_specs=[
        pl.BlockSpec((BN, 32), lambda s, i: (s * NB + i, 0)),
        pl.BlockSpec((BN, 4), lambda s, i: (s * NB + i, 0)),
        pl.BlockSpec((1, 1), lambda s, i: (0, 0)),
    ],
    out_shape=[
        jax.ShapeDtypeStruct((2 * N, 32), jnp.float32),
        jax.ShapeDtypeStruct((2 * N, 4), jnp.float32),
        jax.ShapeDtypeStruct((1, 1), jnp.float32),
    ],
)


_GDN = lax.GatherDimensionNumbers(
    offset_dims=(), collapsed_slice_dims=(0,), start_index_map=(0,))


def _lane_bcast(v, li):
    return lax.gather(v, li, _GDN, (1,),
                      mode=lax.GatherScatterMode.PROMISE_IN_BOUNDS)
def _edge_body(src_h, dst_h, ea, tflat, q8, cvec,
               acch_out, accd_out,
               accH, accD, idx_sg, idx_q1, idx_q2, idx_q3, idx_dst, idx_d2,
               idx_d2b, ea_buf, s0b, s1b, d0b, d1b, htile, den0, den1, cv,
               zbuf):
    cid = lax.axis_index("c")
    sid = lax.axis_index("s")
    cN = cid * N
    base = sid * EPT
    iota = lax.iota(jnp.int32, 16)
    z16 = jnp.zeros((16,), jnp.float32)
    zi = jnp.zeros((16,), jnp.int32)

    @pl.loop(0, CH)
    def _(r):
        htile[r, pl.ds(0, 16)] = z16
        htile[r, pl.ds(16, 16)] = z16

    @pl.loop(0, D_PT // 16)
    def _(i):
        zbuf[pl.ds(i * 16, 16)] = z16

    @pl.loop(0, ROWS_T // CH)
    def _(i):
        pltpu.sync_copy(htile, accH.at[pl.ds(sid * ROWS_T + i * CH, CH)])

    pltpu.sync_copy(zbuf, accD.at[pl.ds(sid * D_PT, D_PT)])

    pltpu.sync_copy(cvec.at[cid], cv)
    plsc.subcore_barrier()
    ce0 = cv[pl.ds(0, 16)]
    ce1 = cv[pl.ds(16, 16)]

    def process_chunk(off, groups, tail=False):
        n_e = groups * 16
        pltpu.sync_copy(src_h.at[pl.ds(off, n_e)], idx_sg.at[pl.ds(0, n_e)])
        pltpu.sync_copy(dst_h.at[pl.ds(off, n_e)], idx_dst.at[pl.ds(0, n_e)])
        pltpu.sync_copy(ea.at[pl.ds(off, n_e)], ea_buf.at[pl.ds(0, n_e)])
        for j in range(groups):
            sl = pl.ds(16 * j, 16)
            d = idx_dst[sl]
            s = idx_sg[sl] + cN
            idx_sg[sl] = s
            idx_q1[sl] = s + 2 * N
            t = d + (cN + 4 * N)
            idx_q2[sl] = t
            idx_q3[sl] = t + 2 * N
            idx_d2[sl] = d * 2
            idx_d2b[sl] = d * 2 + 1
        pltpu.sync_copy(tflat.at[idx_sg], htile)
        pltpu.sync_copy(q8.at[idx_sg], s0b)
        pltpu.sync_copy(q8.at[idx_q1], s1b)
        pltpu.sync_copy(q8.at[idx_q2], d0b)
        pltpu.sync_copy(q8.at[idx_q3], d1b)
        if tail:
            @pl.loop(TAIL, CH)
            def _(r):
                htile[r, pl.ds(0, 16)] = z16
                htile[r, pl.ds(16, 16)] = z16
        for j in range(groups):
            sl = pl.ds(16 * j, 16)
            eav = ea_buf[sl]
            a0 = s0b[sl] + d0b[sl] + eav * ce0
            a1 = s1b[sl] + d1b[sl] + eav * ce1
            a0 = jnp.where(a0 >= 0.0, a0, 0.2 * a0)
            a1 = jnp.where(a1 >= 0.0, a1, 0.2 * a1)
            e0 = jnp.exp(a0)
            e1 = jnp.exp(a1)
            den0[sl] = e0
            den1[sl] = e1
            for l in range(16):
                li = jnp.full((16, 1), l, jnp.int32)
                f0 = _lane_bcast(e0, li)
                f1 = _lane_bcast(e1, li)
                r = 16 * j + l
                htile[r, pl.ds(0, 16)] = htile[r, pl.ds(0, 16)] * f0
                htile[r, pl.ds(16, 16)] = htile[r, pl.ds(16, 16)] * f1

        pltpu.sync_copy(htile, accH.at[idx_dst], add=True)
        pltpu.sync_copy(den0, accD.at[idx_d2], add=True)
        pltpu.sync_copy(den1, accD.at[idx_d2b], add=True)

    for t in range(TAIL // 16, CH // 16):
        sl = pl.ds(16 * t, 16)
        idx_sg[sl] = zi
        idx_q1[sl] = zi
        idx_q2[sl] = zi
        idx_q3[sl] = zi
        idx_dst[sl] = zi
        idx_d2[sl] = zi
        idx_d2b[sl] = zi
        den0[sl] = z16
        den1[sl] = z16

    process_chunk(base + NFULL * CH, TAIL // 16, tail=True)

    @pl.loop(0, NFULL)
    def _(k):
        process_chunk(base + k * CH, CH // 16)

    plsc.subcore_barrier()
    rows = pl.ds(sid * ROWS_T, ROWS_T)
    pltpu.sync_copy(accH.at[rows], acch_out.at[cid, rows])
    pltpu.sync_copy(accD.at[pl.ds(sid * D_PT, D_PT)],
                    accd_out.at[pl.ds(cid * 2 * NPAD + sid * D_PT, D_PT)])


_edge_call = functools.partial(
    pl.kernel,
    out_type=[
        jax.ShapeDtypeStruct((2, NPAD, 32), jnp.float32),
        jax.ShapeDtypeStruct((4 * NPAD,), jnp.float32),
    ],
    mesh=plsc.VectorSubcoreMesh(
        core_axis_name="c", subcore_axis_name="s", num_cores=NC,
        num_subcores=NS),
    compiler_params=pltpu.CompilerParams(use_tc_tiling_on_sc=False),
    scratch_types=[
        pltpu.VMEM_SHARED((NPAD, 32), jnp.float32),
        pltpu.VMEM_SHARED((2 * NPAD,), jnp.float32),
        pltpu.VMEM((CH,), jnp.int32),
        pltpu.VMEM((CH,), jnp.int32),
        pltpu.VMEM((CH,), jnp.int32),
        pltpu.VMEM((CH,), jnp.int32),
        pltpu.VMEM((CH,), jnp.int32),
        pltpu.VMEM((CH,), jnp.int32),
        pltpu.VMEM((CH,), jnp.int32),
        pltpu.VMEM((CH,), jnp.float32),
        pltpu.VMEM((CH,), jnp.float32),
        pltpu.VMEM((CH,), jnp.float32),
        pltpu.VMEM((CH,), jnp.float32),
        pltpu.VMEM((CH,), jnp.float32),
        pltpu.VMEM((CH, 32), jnp.float32),
        pltpu.VMEM((CH,), jnp.float32),
        pltpu.VMEM((CH,), jnp.float32),
        pltpu.VMEM((32,), jnp.float32),
        pltpu.VMEM((D_PT,), jnp.float32),
    ],
)(_edge_body)


def _final_body(acch_ref, accd_ref, t0_ref, t1_ref, q0_ref, q1_ref,
                easum_ref, xres_ref, cf_ref, bias_ref, gamma_ref, beta_ref,
                wout_ref, bout_ref, out_ref, *, last):
    ea_mean = easum_ref[...][0, 0] / E
    a_s = jnp.concatenate([q0_ref[:, 0:2], q1_ref[:, 0:2]], axis=1)
    a_d = jnp.concatenate([q0_ref[:, 2:4], q1_ref[:, 2:4]], axis=1)
    al = a_s + a_d + ea_mean * cf_ref[...][0, :][None, :]
    al = jnp.where(al >= 0.0, al, 0.2 * al)
    exl = jnp.exp(al)
    h = jnp.concatenate([t0_ref[...], t1_ref[...]], axis=1)
    num = jnp.concatenate([acch_ref[0], acch_ref[1]], axis=1)
    den4 = jnp.concatenate([accd_ref[0], accd_ref[1]], axis=1) + exl
    cols = []
    for hh in range(HEADS):
        sl = slice(HD * hh, HD * hh + HD)
        numh = num[:, sl] + h[:, sl] * exl[:, hh:hh + 1]
        cols.append(numh / (den4[:, hh:hh + 1] + 1e-16))
    o = jnp.concatenate(cols, axis=1) + bias_ref[...][0][None, :]
    xo = o + xres_ref[...]
    mu = jnp.mean(xo, axis=1, keepdims=True)
    var = jnp.mean((xo - mu) ** 2, axis=1, keepdims=True)
    xn = (xo - mu) / jnp.sqrt(var + 1e-5) * gamma_ref[...][0][None, :] \
        + beta_ref[...][0][None, :]
    xe = jnp.where(xn > 0.0, xn, jnp.exp(xn) - 1.0)
    if last:
        out_ref[...] = (jnp.dot(xe, wout_ref[...],
                                preferred_element_type=jnp.float32)
                        + bout_ref[...][0][None, :])
    else:
        out_ref[...] = xe


def _make_final(last):
    return pl.pallas_call(
        functools.partial(_final_body, last=last),
        grid=(NB,),
        in_specs=[
            pl.BlockSpec((2, BN, 32), lambda i: (0, i, 0)),
            pl.BlockSpec((2, BN, 2), lambda i: (0, i, 0)),
            pl.BlockSpec((BN, 32), lambda i: (i, 0)),
            pl.BlockSpec((BN, 32), lambda i: (NB + i, 0)),
            pl.BlockSpec((BN, 4), lambda i: (i, 0)),
            pl.BlockSpec((BN, 4), lambda i: (NB + i, 0)),
            pl.BlockSpec((1, 1), lambda i: (0, 0)),
            pl.BlockSpec((BN, HID), lambda i: (i, 0)),
            pl.BlockSpec((1, HEADS), lambda i: (0, 0)),
            pl.BlockSpec((1, HID), lambda i: (0, 0)),
            pl.BlockSpec((1, HID), lambda i: (0, 0)),
            pl.BlockSpec((1, HID), lambda i: (0, 0)),
            pl.BlockSpec((HID, 1), lambda i: (0, 0)),
            pl.BlockSpec((1, 1), lambda i: (0, 0)),
        ],
        out_specs=pl.BlockSpec((BN, 1 if last else HID), lambda i: (i, 0)),
        out_shape=jax.ShapeDtypeStruct((N, 1 if last else HID), jnp.float32),
    )


_final_call = _make_final(False)
_final_call_last = _make_final(True)


def _att_proj(att):
    return (jnp.eye(HEADS, dtype=jnp.float32)[:, None, :]
            * att[:, :, None]).reshape(HID, HEADS)


def kernel(node_features, edge_index, edge_type, edge_attr, W1, att_src1,
           att_dst1, W_edge1, att_edge1, bias1, gamma1, beta1, W2, att_src2,
           att_dst2, W_edge2, att_edge2, bias2, gamma2, beta2, W_out, b_out):
    ea1 = edge_attr[:, 0]
    ea2d = ea1.reshape(800, 1000)

    def layer(x, W, att_src, att_dst, W_edge, att_edge, bias, gamma, beta,
              last, W_out=None, b_out=None):
        cv = jnp.sum(W_edge.reshape(HEADS, HD) * att_edge, axis=1)
        cvec_sc = jnp.repeat(cv.reshape(2, 2), 16, axis=1)
        cf = cv.reshape(1, HEADS)
        tflat, qflat, easum = _prep_call(
            x, W, _att_proj(att_src), _att_proj(att_dst), ea2d)
        q8 = qflat.T.reshape(-1)
        acch, accd = _edge_call(edge_index[0], edge_index[1], ea1, tflat,
                                q8, cvec_sc)
        acch = acch[:, :N, :]
        accd = accd.reshape(2, 2 * NPAD)[:, :2 * N].reshape(2, N, 2)
        fin = _final_call_last if last else _final_call
        if W_out is None:
            W_out = jnp.zeros((HID, 1), jnp.float32)
            b_out = jnp.zeros((1, 1), jnp.float32)
        return fin(acch, accd, tflat, tflat, qflat, qflat, easum, x, cf,
                   bias.reshape(1, HID), gamma.reshape(1, HID),
                   beta.reshape(1, HID), W_out, b_out)

    x1 = layer(node_features, W1, att_src1, att_dst1, W_edge1, att_edge1,
               bias1, gamma1, beta1, last=False)
    y = layer(x1, W2, att_src2, att_dst2, W_edge2, att_edge2,
              bias2, gamma2, beta2, last=True,
              W_out=W_out, b_out=b_out.reshape(1, 1))
    return y


# speedup vs baseline: 90.4674x; 2.3655x over previous
import functools

import jax
import jax.numpy as jnp
from jax import lax
from jax.experimental import pallas as pl
from jax.experimental.pallas import tpu as pltpu
from jax.experimental.pallas import tpu_sc as plsc

N = 50000
E = 800000
HEADS = 4
HD = 16
HID = 64

NC = 2
NS = 16
EPT = E // NS
CH = 128
NFULL = EPT // CH
TAIL = EPT - NFULL * CH
NPAD = 51200
ROWS_T = NPAD // NS
D_PT = 2 * NPAD // NS

BN = 2000
NB = N // BN


def _prep_body(x_ref, w_ref, asrc_ref, adst_ref, ea_ref, t_ref, q_ref,
               easum_ref):
    s = pl.program_id(0)
    i = pl.program_id(1)
    h = jnp.dot(x_ref[...], w_ref[...], preferred_element_type=jnp.float32)
    a_s = jnp.dot(h, asrc_ref[...], preferred_element_type=jnp.float32)
    a_d = jnp.dot(h, adst_ref[...], preferred_element_type=jnp.float32)
    t_ref[...] = jnp.where(s == 0, h[:, :32], h[:, 32:])
    q0 = jnp.concatenate([a_s[:, 0:2], a_d[:, 0:2]], axis=1)
    q1 = jnp.concatenate([a_s[:, 2:4], a_d[:, 2:4]], axis=1)
    q_ref[...] = jnp.where(s == 0, q0, q1)

    @pl.when((s == 0) & (i == 0))
    def _():
        easum_ref[...] = jnp.zeros_like(easum_ref)

    @pl.when(s == 0)
    def _():
        easum_ref[...] += jnp.sum(ea_ref[...])[None, None]


_prep_call = pl.pallas_call(
    _prep_body,
    grid=(2, NB),
    in_specs=[
        pl.BlockSpec((BN, HID), lambda s, i: (i, 0)),
        pl.BlockSpec((HID, HID), lambda s, i: (0, 0)),
        pl.BlockSpec((HID, HEADS), lambda s, i: (0, 0)),
        pl.BlockSpec((HID, HEADS), lambda s, i: (0, 0)),
        pl.BlockSpec((800 // NB, 1000), lambda s, i: (i, 0)),
    ],
    out_specs=[
        pl.BlockSpec((BN, 32), lambda s, i: (s * NB + i, 0)),
        pl.BlockSpec((BN, 4), lambda s, i: (s * NB + i, 0)),
        pl.BlockSpec((1, 1), lambda s, i: (0, 0)),
    ],
    out_shape=[
        jax.ShapeDtypeStruct((2 * N, 32), jnp.float32),
        jax.ShapeDtypeStruct((2 * N, 4), jnp.float32),
        jax.ShapeDtypeStruct((1, 1), jnp.float32),
    ],
)


_GDN = lax.GatherDimensionNumbers(
    offset_dims=(), collapsed_slice_dims=(0,), start_index_map=(0,))


def _lane_bcast(v, li):
    return lax.gather(v, li, _GDN, (1,),
                      mode=lax.GatherScatterMode.PROMISE_IN_BOUNDS)
def _edge_body(src_h, dst_h, ea, tflat, q8, cvec,
               acch_out, accd_out,
               accH, accD, cv, zbuf, *bufs):
    setA = bufs[0:15]
    setB = bufs[15:30]
    sem_lin = bufs[30:32]
    sem_gat = bufs[32:34]
    sem_scat = bufs[34:36]
    sets = (setA, setB)
    cid = lax.axis_index("c")
    sid = lax.axis_index("s")
    cN = cid * N
    base = sid * EPT
    iota = lax.iota(jnp.int32, 16)
    z16 = jnp.zeros((16,), jnp.float32)
    zi = jnp.zeros((16,), jnp.int32)

    htile0 = setA[12]

    @pl.loop(0, CH)
    def _(r):
        htile0[r, pl.ds(0, 16)] = z16
        htile0[r, pl.ds(16, 16)] = z16

    @pl.loop(0, D_PT // 16)
    def _(i):
        zbuf[pl.ds(i * 16, 16)] = z16

    @pl.loop(0, ROWS_T // CH)
    def _(i):
        pltpu.sync_copy(htile0, accH.at[pl.ds(sid * ROWS_T + i * CH, CH)])

    pltpu.sync_copy(zbuf, accD.at[pl.ds(sid * D_PT, D_PT)])

    pltpu.sync_copy(cvec.at[cid], cv)
    plsc.subcore_barrier()
    ce0 = cv[pl.ds(0, 16)]
    ce1 = cv[pl.ds(16, 16)]

    def lin_copies(off, S, n_e):
        (idx_sg, idx_q1, idx_q2, idx_q3, idx_dst, idx_d2, idx_d2b, ea_buf,
         s0b, s1b, d0b, d1b, htile, den0, den1) = sets[S]
        return (
            (src_h.at[pl.ds(off, n_e)], idx_sg.at[pl.ds(0, n_e)]),
            (dst_h.at[pl.ds(off, n_e)], idx_dst.at[pl.ds(0, n_e)]),
            (ea.at[pl.ds(off, n_e)], ea_buf.at[pl.ds(0, n_e)]),
        )

    def issue_lin(off, S, n_e=CH):
        for s_, d_ in lin_copies(off, S, n_e):
            pltpu.async_copy(s_, d_, sem_lin[S])

    def wait_lin(off, S, n_e=CH):
        for s_, d_ in lin_copies(off, S, n_e):
            pltpu.make_async_copy(s_, d_, sem_lin[S]).wait()

    def offsets(S, groups=CH // 16):
        (idx_sg, idx_q1, idx_q2, idx_q3, idx_dst, idx_d2, idx_d2b, ea_buf,
         s0b, s1b, d0b, d1b, htile, den0, den1) = sets[S]
        for j in range(groups):
            sl = pl.ds(16 * j, 16)
            d = idx_dst[sl]
            s = idx_sg[sl] + cN
            idx_sg[sl] = s
            idx_q1[sl] = s + 2 * N
            t = d + (cN + 4 * N)
            idx_q2[sl] = t
            idx_q3[sl] = t + 2 * N
            idx_d2[sl] = d * 2
            idx_d2b[sl] = d * 2 + 1

    def gat_copies(S):
        (idx_sg, idx_q1, idx_q2, idx_q3, idx_dst, idx_d2, idx_d2b, ea_buf,
         s0b, s1b, d0b, d1b, htile, den0, den1) = sets[S]
        return (
            (tflat.at[idx_sg], htile),
            (q8.at[idx_sg], s0b),
            (q8.at[idx_q1], s1b),
            (q8.at[idx_q2], d0b),
            (q8.at[idx_q3], d1b),
        )

    def issue_gat(S):
        for s_, d_ in gat_copies(S):
            pltpu.async_copy(s_, d_, sem_gat[S])

    def wait_gat(S):
        for s_, d_ in gat_copies(S):
            pltpu.make_async_copy(s_, d_, sem_gat[S]).wait()

    def scat_copies(S):
        (idx_sg, idx_q1, idx_q2, idx_q3, idx_dst, idx_d2, idx_d2b, ea_buf,
         s0b, s1b, d0b, d1b, htile, den0, den1) = sets[S]
        return (
            (htile, accH.at[idx_dst]),
            (den0, accD.at[idx_d2]),
            (den1, accD.at[idx_d2b]),
        )

    def issue_scat(S):
        for s_, d_ in scat_copies(S):
            pltpu.async_copy(s_, d_, sem_scat[S], add=True)

    def wait_scat(S):
        for s_, d_ in scat_copies(S):
            pltpu.make_async_copy(s_, d_, sem_scat[S]).wait()

    def compute(S, groups=CH // 16, tail=False):
        (idx_sg, idx_q1, idx_q2, idx_q3, idx_dst, idx_d2, idx_d2b, ea_buf,
         s0b, s1b, d0b, d1b, htile, den0, den1) = sets[S]
        if tail:
            @pl.loop(TAIL, CH)
            def _(r):
                htile[r, pl.ds(0, 16)] = z16
                htile[r, pl.ds(16, 16)] = z16
        for j in range(groups):
            sl = pl.ds(16 * j, 16)
            eav = ea_buf[sl]
            a0 = s0b[sl] + d0b[sl] + eav * ce0
            a1 = s1b[sl] + d1b[sl] + eav * ce1
            a0 = jnp.where(a0 >= 0.0, a0, 0.2 * a0)
            a1 = jnp.where(a1 >= 0.0, a1, 0.2 * a1)
            e0 = jnp.exp(a0)
            e1 = jnp.exp(a1)
            den0[sl] = e0
            den1[sl] = e1
            for l in range(16):
                li = jnp.full((16, 1), l, jnp.int32)
                f0 = _lane_bcast(e0, li)
                f1 = _lane_bcast(e1, li)
                r = 16 * j + l
                htile[r, pl.ds(0, 16)] = htile[r, pl.ds(0, 16)] * f0
                htile[r, pl.ds(16, 16)] = htile[r, pl.ds(16, 16)] * f1

    (idx_sg, idx_q1, idx_q2, idx_q3, idx_dst, idx_d2, idx_d2b, ea_buf,
     s0b, s1b, d0b, d1b, htile, den0, den1) = setA
    for t in range(TAIL // 16, CH // 16):
        sl = pl.ds(16 * t, 16)
        idx_sg[sl] = zi
        idx_q1[sl] = zi
        idx_q2[sl] = zi
        idx_q3[sl] = zi
        idx_dst[sl] = zi
        idx_d2[sl] = zi
        idx_d2b[sl] = zi
        den0[sl] = z16
        den1[sl] = z16
    tail_off = base + NFULL * CH
    issue_lin(tail_off, 0, TAIL)
    wait_lin(tail_off, 0, TAIL)
    offsets(0, TAIL // 16)
    issue_gat(0)
    wait_gat(0)
    compute(0, TAIL // 16, tail=True)
    issue_scat(0)
    wait_scat(0)

    issue_lin(base, 0)
    wait_lin(base, 0)
    offsets(0)
    issue_gat(0)
    issue_lin(base + CH, 1)

    @pl.loop(0, NFULL // 2)
    def _(p):
        k0 = 2 * p
        wait_gat(0)
        compute(0)
        issue_scat(0)
        wait_lin(base + (k0 + 1) * CH, 1)
        offsets(1)
        issue_gat(1)
        wait_scat(0)

        @pl.when(p < NFULL // 2 - 1)
        def _():
            issue_lin(base + (k0 + 2) * CH, 0)

        wait_gat(1)
        compute(1)
        issue_scat(1)

        @pl.when(p < NFULL // 2 - 1)
        def _():
            wait_lin(base + (k0 + 2) * CH, 0)
            offsets(0)
            issue_gat(0)
        wait_scat(1)

        @pl.when(p < NFULL // 2 - 1)
        def _():
            issue_lin(base + (k0 + 3) * CH, 1)

    plsc.subcore_barrier()
    rows = pl.ds(sid * ROWS_T, ROWS_T)
    pltpu.sync_copy(accH.at[rows], acch_out.at[cid, rows])
    pltpu.sync_copy(accD.at[pl.ds(sid * D_PT, D_PT)],
                    accd_out.at[pl.ds(cid * 2 * NPAD + sid * D_PT, D_PT)])


_edge_call = functools.partial(
    pl.kernel,
    out_type=[
        jax.ShapeDtypeStruct((2, NPAD, 32), jnp.float32),
        jax.ShapeDtypeStruct((4 * NPAD,), jnp.float32),
    ],
    mesh=plsc.VectorSubcoreMesh(
        core_axis_name="c", subcore_axis_name="s", num_cores=NC,
        num_subcores=NS),
    compiler_params=pltpu.CompilerParams(use_tc_tiling_on_sc=False),
    scratch_types=[
        pltpu.VMEM_SHARED((NPAD, 32), jnp.float32),
        pltpu.VMEM_SHARED((2 * NPAD,), jnp.float32),
        pltpu.VMEM((32,), jnp.float32),
        pltpu.VMEM((D_PT,), jnp.float32),
    ] + 2 * [
        pltpu.VMEM((CH,), jnp.int32),
        pltpu.VMEM((CH,), jnp.int32),
        pltpu.VMEM((CH,), jnp.int32),
        pltpu.VMEM((CH,), jnp.int32),
        pltpu.VMEM((CH,), jnp.int32),
        pltpu.VMEM((CH,), jnp.int32),
        pltpu.VMEM((CH,), jnp.int32),
        pltpu.VMEM((CH,), jnp.float32),
        pltpu.VMEM((CH,), jnp.float32),
        pltpu.VMEM((CH,), jnp.float32),
        pltpu.VMEM((CH,), jnp.float32),
        pltpu.VMEM((CH,), jnp.float32),
        pltpu.VMEM((CH, 32), jnp.float32),
        pltpu.VMEM((CH,), jnp.float32),
        pltpu.VMEM((CH,), jnp.float32),
    ] + 6 * [pltpu.SemaphoreType.DMA],
)(_edge_body)


def _final_body(acch_ref, accd_ref, t0_ref, t1_ref, q0_ref, q1_ref,
                easum_ref, xres_ref, cf_ref, bias_ref, gamma_ref, beta_ref,
                wout_ref, bout_ref, out_ref, *, last):
    ea_mean = easum_ref[...][0, 0] / E
    a_s = jnp.concatenate([q0_ref[:, 0:2], q1_ref[:, 0:2]], axis=1)
    a_d = jnp.concatenate([q0_ref[:, 2:4], q1_ref[:, 2:4]], axis=1)
    al = a_s + a_d + ea_mean * cf_ref[...][0, :][None, :]
    al = jnp.where(al >= 0.0, al, 0.2 * al)
    exl = jnp.exp(al)
    h = jnp.concatenate([t0_ref[...], t1_ref[...]], axis=1)
    num = jnp.concatenate([acch_ref[0], acch_ref[1]], axis=1)
    den4 = jnp.concatenate([accd_ref[0], accd_ref[1]], axis=1) + exl
    cols = []
    for hh in range(HEADS):
        sl = slice(HD * hh, HD * hh + HD)
        numh = num[:, sl] + h[:, sl] * exl[:, hh:hh + 1]
        cols.append(numh / (den4[:, hh:hh + 1] + 1e-16))
    o = jnp.concatenate(cols, axis=1) + bias_ref[...][0][None, :]
    xo = o + xres_ref[...]
    mu = jnp.mean(xo, axis=1, keepdims=True)
    var = jnp.mean((xo - mu) ** 2, axis=1, keepdims=True)
    xn = (xo - mu) / jnp.sqrt(var + 1e-5) * gamma_ref[...][0][None, :] \
        + beta_ref[...][0][None, :]
    xe = jnp.where(xn > 0.0, xn, jnp.exp(xn) - 1.0)
    if last:
        out_ref[...] = (jnp.dot(xe, wout_ref[...],
                                preferred_element_type=jnp.float32)
                        + bout_ref[...][0][None, :])
    else:
        out_ref[...] = xe


def _make_final(last):
    return pl.pallas_call(
        functools.partial(_final_body, last=last),
        grid=(NB,),
        in_specs=[
            pl.BlockSpec((2, BN, 32), lambda i: (0, i, 0)),
            pl.BlockSpec((2, BN, 2), lambda i: (0, i, 0)),
            pl.BlockSpec((BN, 32), lambda i: (i, 0)),
            pl.BlockSpec((BN, 32), lambda i: (NB + i, 0)),
            pl.BlockSpec((BN, 4), lambda i: (i, 0)),
            pl.BlockSpec((BN, 4), lambda i: (NB + i, 0)),
            pl.BlockSpec((1, 1), lambda i: (0, 0)),
            pl.BlockSpec((BN, HID), lambda i: (i, 0)),
            pl.BlockSpec((1, HEADS), lambda i: (0, 0)),
            pl.BlockSpec((1, HID), lambda i: (0, 0)),
            pl.BlockSpec((1, HID), lambda i: (0, 0)),
            pl.BlockSpec((1, HID), lambda i: (0, 0)),
            pl.BlockSpec((HID, 1), lambda i: (0, 0)),
            pl.BlockSpec((1, 1), lambda i: (0, 0)),
        ],
        out_specs=pl.BlockSpec((BN, 1 if last else HID), lambda i: (i, 0)),
        out_shape=jax.ShapeDtypeStruct((N, 1 if last else HID), jnp.float32),
    )


_final_call = _make_final(False)
_final_call_last = _make_final(True)


def _att_proj(att):
    return (jnp.eye(HEADS, dtype=jnp.float32)[:, None, :]
            * att[:, :, None]).reshape(HID, HEADS)


def kernel(node_features, edge_index, edge_type, edge_attr, W1, att_src1,
           att_dst1, W_edge1, att_edge1, bias1, gamma1, beta1, W2, att_src2,
           att_dst2, W_edge2, att_edge2, bias2, gamma2, beta2, W_out, b_out):
    ea1 = edge_attr[:, 0]
    ea2d = ea1.reshape(800, 1000)

    def layer(x, W, att_src, att_dst, W_edge, att_edge, bias, gamma, beta,
              last, W_out=None, b_out=None):
        cv = jnp.sum(W_edge.reshape(HEADS, HD) * att_edge, axis=1)
        cvec_sc = jnp.repeat(cv.reshape(2, 2), 16, axis=1)
        cf = cv.reshape(1, HEADS)
        tflat, qflat, easum = _prep_call(
            x, W, _att_proj(att_src), _att_proj(att_dst), ea2d)
        q8 = qflat.T.reshape(-1)
        acch, accd = _edge_call(edge_index[0], edge_index[1], ea1, tflat,
                                q8, cvec_sc)
        acch = acch[:, :N, :]
        accd = accd.reshape(2, 2 * NPAD)[:, :2 * N].reshape(2, N, 2)
        fin = _final_call_last if last else _final_call
        if W_out is None:
            W_out = jnp.zeros((HID, 1), jnp.float32)
            b_out = jnp.zeros((1, 1), jnp.float32)
        return fin(acch, accd, tflat, tflat, qflat, qflat, easum, x, cf,
                   bias.reshape(1, HID), gamma.reshape(1, HID),
                   beta.reshape(1, HID), W_out, b_out)

    x1 = layer(node_features, W1, att_src1, att_dst1, W_edge1, att_edge1,
               bias1, gamma1, beta1, last=False)
    y = layer(x1, W2, att_src2, att_dst2, W_edge2, att_edge2,
              bias2, gamma2, beta2, last=True,
              W_out=W_out, b_out=b_out.reshape(1, 1))
    return y
